# baseline (speedup 1.0000x reference)
import jax
import jax.numpy as jnp
from jax import lax
from jax.experimental import pallas as pl
from jax.experimental.pallas import tpu as pltpu
from jax.experimental.pallas import tpu_sc as plsc

_N = 10000
_D = 128
_E = 320000
_NC = 2
_NS = 16
_NW = _NC * _NS
_CHUNK = 128
_NBUF = 2
_CPW = 80
_HALF = _CPW // 2
_E_PAD = _NW * _CPW * _CHUNK
_ROWS_PER_TILE = 632
_N_PAD = _ROWS_PER_TILE * _NS


def _make_spmm(with_deg: bool):
    mesh = plsc.VectorSubcoreMesh(
        core_axis_name="c", subcore_axis_name="s",
        num_cores=_NC, num_subcores=_NS)
    out_type = [jax.ShapeDtypeStruct((_NC, _N_PAD, _D), jnp.float32)]
    if with_deg:
        out_type.append(jax.ShapeDtypeStruct((_NC, _N_PAD), jnp.float32))
    scratch = [
        pltpu.VMEM((_HALF, _CHUNK), jnp.int32),
        pltpu.VMEM((_HALF, _CHUNK), jnp.int32),
        [pltpu.VMEM((_CHUNK, _D), jnp.float32) for _ in range(_NBUF)],
        pltpu.VMEM((_CHUNK,), jnp.float32),
        pltpu.VMEM_SHARED((_N_PAD, _D), jnp.float32),
        pltpu.VMEM_SHARED((_N_PAD,), jnp.float32),
        [pltpu.SemaphoreType.DMA for _ in range(_NBUF)],
        [pltpu.SemaphoreType.DMA for _ in range(_NBUF)],
        pltpu.SemaphoreType.DMA,
    ]

    def body(h_hbm, z2d_hbm, z1d_hbm, cols_hbm, rows_hbm, *rest):
        if with_deg:
            (out_acc, out_deg, idx_c, idx_r, bufs, ones_v, acc, dacc,
             sem_g, sem_s, sem_d) = rest
        else:
            out_deg = None
            (out_acc, idx_c, idx_r, bufs, ones_v, acc, dacc,
             sem_g, sem_s, sem_d) = rest
        cid = lax.axis_index("c")
        sid = lax.axis_index("s")
        wid = sid * _NC + cid

        pieces = []
        r0 = 0
        while r0 < _ROWS_PER_TILE:
            pieces.append((r0, min(_CHUNK, _ROWS_PER_TILE - r0)))
            r0 += _CHUNK
        for k, (r0, rk) in enumerate(pieces):
            b = bufs[k % 2]
            pltpu.sync_copy(z2d_hbm.at[pl.ds(0, rk)], b.at[pl.ds(0, rk)])
            pltpu.sync_copy(
                b.at[pl.ds(0, rk)],
                acc.at[pl.ds(sid * _ROWS_PER_TILE + r0, rk)])
        if with_deg:
            @pl.when(sid == 0)
            def _():
                pltpu.sync_copy(z1d_hbm, dacc)
            for i in range(_CHUNK // 16):
                ones_v[pl.ds(i * 16, 16)] = jnp.ones((16,), jnp.float32)
        plsc.subcore_barrier()

        def fire_gather(c, b):
            pltpu.async_copy(h_hbm.at[idx_c.at[c]], bufs[b], sem_g[b])
            if with_deg:
                pltpu.async_copy(ones_v, dacc.at[idx_r.at[c]], sem_d,
                                 add=True)

        def drain_gather(b):
            pltpu.make_async_copy(
                h_hbm.at[idx_c.at[0]], bufs[b], sem_g[b]).wait()
            if with_deg:
                pltpu.make_async_copy(
                    ones_v, dacc.at[idx_r.at[0]], sem_d).wait()

        def drain_scatter(b):
            pltpu.make_async_copy(
                bufs[b], acc.at[idx_r.at[0]], sem_s[b]).wait()

        n_pairs = _HALF // 2

        def step(p, carry):
            c0 = 2 * p

            @pl.when(p > 0)
            def _():
                drain_scatter(1)
            fire_gather(c0 + 1, 1)
            drain_gather(0)
            pltpu.async_copy(
                bufs[0], acc.at[idx_r.at[c0]], sem_s[0], add=True)
            drain_gather(1)
            pltpu.async_copy(
                bufs[1], acc.at[idx_r.at[c0 + 1]], sem_s[1], add=True)
            drain_scatter(0)

            @pl.when(p < n_pairs - 1)
            def _():
                fire_gather(c0 + 2, 0)
            return carry

        for half in range(2):
            pltpu.sync_copy(
                cols_hbm.at[wid, pl.ds(half * _HALF, _HALF)], idx_c)
            pltpu.sync_copy(
                rows_hbm.at[wid, pl.ds(half * _HALF, _HALF)], idx_r)
            fire_gather(0, 0)
            lax.fori_loop(0, n_pairs, step, 0)
            drain_scatter(1)
        plsc.subcore_barrier()

        for k, (r0, rk) in enumerate(pieces):
            b = bufs[k % 2]
            sl = pl.ds(sid * _ROWS_PER_TILE + r0, rk)
            pltpu.sync_copy(acc.at[sl], b.at[pl.ds(0, rk)])
            pltpu.sync_copy(b.at[pl.ds(0, rk)], out_acc.at[cid, sl])
        if with_deg:
            @pl.when(sid == 0)
            def _():
                pltpu.sync_copy(dacc, out_deg.at[cid])

    return pl.kernel(body, out_type=out_type, mesh=mesh,
                     scratch_types=scratch)


_spmm_deg = _make_spmm(with_deg=True)
_spmm_nodeg = _make_spmm(with_deg=False)


def _tc1_body(acc_ref, deg_ref, x_ref, wn_ref, wr_ref, h_ref, inv_ref):
    deg = jnp.maximum(deg_ref[0, :_N] + deg_ref[1, :_N], 1.0)
    inv = 1.0 / deg
    agg = (acc_ref[0, :_N] + acc_ref[1, :_N]) * inv
    h = (jnp.dot(agg, wn_ref[...], preferred_element_type=jnp.float32)
         + jnp.dot(x_ref[...], wr_ref[...], preferred_element_type=jnp.float32))
    h_ref[...] = jnp.maximum(h, 0.0)
    inv_ref[...] = inv


def _tc2_body(acc_ref, inv_ref, h_ref, wn_ref, wr_ref, out_ref):
    agg = (acc_ref[0, :_N] + acc_ref[1, :_N]) * inv_ref[...]
    out_ref[...] = (
        jnp.dot(agg, wn_ref[...], preferred_element_type=jnp.float32)
        + jnp.dot(h_ref[...], wr_ref[...], preferred_element_type=jnp.float32))


_tc1 = pl.pallas_call(
    _tc1_body,
    out_shape=[jax.ShapeDtypeStruct((_N, _D), jnp.float32),
               jax.ShapeDtypeStruct((_N, 1), jnp.float32)])

_tc2 = pl.pallas_call(
    _tc2_body,
    out_shape=jax.ShapeDtypeStruct((_N, _D), jnp.float32))


def kernel(x, edge_index, W_neigh1, W_root1, W_neigh2, W_root2):
    rows = edge_index[0].astype(jnp.int32)
    cols = edge_index[1].astype(jnp.int32)
    npad = _E_PAD - _E
    pad_ar = jnp.arange(npad, dtype=jnp.int32)
    cols_p = jnp.concatenate(
        [cols, pad_ar % _N]).reshape(_NW, _CPW, _CHUNK)
    rows_p = jnp.concatenate(
        [rows, _N + pad_ar % (_N_PAD - _N)]).reshape(_NW, _CPW, _CHUNK)
    z2d = jnp.zeros((_ROWS_PER_TILE, _D), jnp.float32)
    z1d = jnp.zeros((_N_PAD,), jnp.float32)

    acc1, deg = _spmm_deg(x, z2d, z1d, cols_p, rows_p)
    deg = deg.reshape(_NC, _N_PAD, 1)
    h, inv = _tc1(acc1, deg, x, W_neigh1.T, W_root1.T)
    (acc2,) = _spmm_nodeg(h, z2d, z1d, cols_p, rows_p)
    return _tc2(acc2, inv, h, W_neigh2.T, W_root2.T)
